# baseline (speedup 1.0000x reference)
import functools

import jax
import jax.numpy as jnp
from jax import lax
from jax.experimental import pallas as pl
from jax.experimental.pallas import tpu as pltpu
from jax.experimental.pallas import tpu_sc as plsc

N = 10000
E = 320000
F = 128
CH = 128
NC = 2
NS = 16
NW = NC * NS
NCHT = 80
NCHUNK = NW * NCHT
E_PAD = NCHUNK * CH
DUMMY = 240
ACC_N = N + DUMMY
RPT = ACC_N // NS
BB = 16
ZR = 64


def _make_agg():
    mesh = plsc.VectorSubcoreMesh(core_axis_name="c", subcore_axis_name="s")

    @functools.partial(
        pl.kernel,
        mesh=mesh,
        out_type=jax.ShapeDtypeStruct((NC, ACC_N, F), jnp.float32),
        scratch_types=[
            pltpu.VMEM((BB, CH), jnp.int32),
            pltpu.VMEM((BB, CH), jnp.int32),
            pltpu.VMEM((CH, F), jnp.float32),
            pltpu.VMEM((CH, F), jnp.float32),
            pltpu.VMEM((ZR, F), jnp.float32),
            pltpu.VMEM_SHARED((ACC_N, F), jnp.float32),
            pltpu.SemaphoreType.DMA,
            pltpu.SemaphoreType.DMA,
        ],
    )
    def agg(y_hbm, src_hbm, dst_hbm, part_hbm,
            sidx, didx, rows0, rows1, zbuf, acc, sem0, sem1):
        c = lax.axis_index("c")
        s = lax.axis_index("s")
        wid = s * NC + c
        c0 = pl.multiple_of(wid * NCHT, 8)

        def zstore(k, carry):
            i = k // (F // 16)
            j = (k % (F // 16)) * 16
            zbuf[i, pl.ds(j, 16)] = jnp.zeros((16,), jnp.float32)
            return carry

        lax.fori_loop(0, ZR * (F // 16), zstore, 0)
        r0 = pl.multiple_of(s * RPT, 8)

        def zcopy(i, carry):
            pltpu.sync_copy(zbuf, acc.at[pl.ds(r0 + i * ZR, ZR)])
            return carry

        lax.fori_loop(0, RPT // ZR, zcopy, 0)
        plsc.subcore_barrier()

        def batch(b, carry):
            cb = pl.multiple_of(c0 + b * BB, 8)
            pltpu.sync_copy(src_hbm.at[pl.ds(cb, BB)], sidx)
            pltpu.sync_copy(dst_hbm.at[pl.ds(cb, BB)], didx)
            pltpu.async_copy(y_hbm.at[sidx.at[0]], rows0, sem0)

            def pair(j, carry2):
                k0 = 2 * j
                pltpu.async_copy(y_hbm.at[sidx.at[k0 + 1]], rows1, sem1)
                pltpu.make_async_copy(
                    y_hbm.at[sidx.at[k0]], rows0, sem0).wait()
                pltpu.sync_copy(rows0, acc.at[didx.at[k0]], add=True)

                @pl.when(j < (BB // 2) - 1)
                def _next_even():
                    pltpu.async_copy(y_hbm.at[sidx.at[k0 + 2]], rows0, sem0)

                pltpu.make_async_copy(
                    y_hbm.at[sidx.at[k0 + 1]], rows1, sem1).wait()
                pltpu.sync_copy(rows1, acc.at[didx.at[k0 + 1]], add=True)
                return carry2

            lax.fori_loop(0, BB // 2, pair, 0)
            return carry

        lax.fori_loop(0, NCHT // BB, batch, 0)
        plsc.subcore_barrier()

        pltpu.sync_copy(acc.at[pl.ds(r0, RPT)], part_hbm.at[c, pl.ds(r0, RPT)])

    return agg


_AGG = None


def _get_agg():
    global _AGG
    if _AGG is None:
        _AGG = _make_agg()
    return _AGG


BM = 1000


def _matmul(x, w):
    m, k = x.shape
    n = w.shape[1]

    def body(x_ref, w_ref, o_ref):
        o_ref[...] = jnp.dot(x_ref[...], w_ref[...],
                             preferred_element_type=jnp.float32)

    return pl.pallas_call(
        body,
        grid=(m // BM,),
        in_specs=[pl.BlockSpec((BM, k), lambda i: (i, 0)),
                  pl.BlockSpec((k, n), lambda i: (0, 0))],
        out_specs=pl.BlockSpec((BM, n), lambda i: (i, 0)),
        out_shape=jax.ShapeDtypeStruct((m, n), jnp.float32),
    )(x, w)


def _mid(y, parts, b, w):
    m, n = y.shape

    def body(y_ref, p_ref, b_ref, w_ref, o_ref):
        h = y_ref[...] + p_ref[0] + p_ref[1] + b_ref[...]
        h = jnp.maximum(h, 0.0)
        o_ref[...] = jnp.dot(h, w_ref[...], preferred_element_type=jnp.float32)

    return pl.pallas_call(
        body,
        grid=(m // BM,),
        in_specs=[pl.BlockSpec((BM, n), lambda i: (i, 0)),
                  pl.BlockSpec((NC, BM, n), lambda i: (0, i, 0)),
                  pl.BlockSpec((1, n), lambda i: (0, 0)),
                  pl.BlockSpec((n, n), lambda i: (0, 0))],
        out_specs=pl.BlockSpec((BM, n), lambda i: (i, 0)),
        out_shape=jax.ShapeDtypeStruct((m, n), jnp.float32),
    )(y, parts, b.reshape(1, n), w)


def _final(y, parts, b):
    m, n = y.shape

    def body(y_ref, p_ref, b_ref, o_ref):
        o_ref[...] = jnp.maximum(
            y_ref[...] + p_ref[0] + p_ref[1] + b_ref[...], 0.0)

    return pl.pallas_call(
        body,
        grid=(m // BM,),
        in_specs=[pl.BlockSpec((BM, n), lambda i: (i, 0)),
                  pl.BlockSpec((NC, BM, n), lambda i: (0, i, 0)),
                  pl.BlockSpec((1, n), lambda i: (0, 0))],
        out_specs=pl.BlockSpec((BM, n), lambda i: (i, 0)),
        out_shape=jax.ShapeDtypeStruct((m, n), jnp.float32),
    )(y, parts, b.reshape(1, n))


def kernel(x, edge_index, W1, b1, W2, b2):
    npad = E_PAD - E
    src = jnp.concatenate(
        [edge_index[0], jnp.zeros((npad,), jnp.int32)]).reshape(NCHUNK, CH)
    dst = jnp.concatenate(
        [edge_index[1],
         N + (jnp.arange(npad, dtype=jnp.int32) % DUMMY)]).reshape(NCHUNK, CH)
    agg = _get_agg()
    y1 = _matmul(x, W1)
    p1 = agg(y1, src, dst)
    y2 = _mid(y1, p1, b1, W2)
    p2 = agg(y2, src, dst)
    return _final(y2, p2, b2)


# speedup vs baseline: 3.0308x; 3.0308x over previous
import functools

import jax
import jax.numpy as jnp
from jax import lax
from jax.experimental import pallas as pl
from jax.experimental.pallas import tpu as pltpu
from jax.experimental.pallas import tpu_sc as plsc

N = 10000
E = 320000
F = 128
CH = 128
NC = 2
NS = 16
NW = NC * NS
NCHT = 80
NCHUNK = NW * NCHT
E_PAD = NCHUNK * CH
DUMMY = 240
ACC_N = N + DUMMY
RPT = ACC_N // NS
BB = 16
ZR = 64


def _make_agg():
    mesh = plsc.VectorSubcoreMesh(core_axis_name="c", subcore_axis_name="s")

    @functools.partial(
        pl.kernel,
        mesh=mesh,
        out_type=jax.ShapeDtypeStruct((NC, ACC_N, F), jnp.float32),
        scratch_types=[
            pltpu.VMEM((BB, CH), jnp.int32),
            pltpu.VMEM((BB, CH), jnp.int32),
            pltpu.VMEM((CH, F), jnp.float32),
            pltpu.VMEM((CH, F), jnp.float32),
            pltpu.VMEM((ZR, F), jnp.float32),
            pltpu.VMEM_SHARED((ACC_N, F), jnp.float32),
            pltpu.SemaphoreType.DMA,
            pltpu.SemaphoreType.DMA,
        ],
    )
    def agg(y_hbm, src_hbm, dst_hbm, part_hbm,
            sidx, didx, rows0, rows1, zbuf, acc, sem0, sem1):
        c = lax.axis_index("c")
        s = lax.axis_index("s")
        wid = s * NC + c
        c0 = pl.multiple_of(wid * NCHT, 8)

        def zstore(k, carry):
            i = k // (F // 16)
            j = (k % (F // 16)) * 16
            zbuf[i, pl.ds(j, 16)] = jnp.zeros((16,), jnp.float32)
            return carry

        lax.fori_loop(0, ZR * (F // 16), zstore, 0)
        r0 = pl.multiple_of(s * RPT, 8)

        def zcopy(i, carry):
            pltpu.sync_copy(zbuf, acc.at[pl.ds(r0 + i * ZR, ZR)])
            return carry

        lax.fori_loop(0, RPT // ZR, zcopy, 0)
        plsc.subcore_barrier()

        def batch(b, carry):
            cb = pl.multiple_of(c0 + b * BB, 8)
            pltpu.sync_copy(src_hbm.at[pl.ds(cb, BB)], sidx)
            pltpu.sync_copy(dst_hbm.at[pl.ds(cb, BB)], didx)
            pltpu.async_copy(y_hbm.at[sidx.at[0]], rows0, sem0)

            def pair(j, carry2):
                k0 = 2 * j
                pltpu.async_copy(y_hbm.at[sidx.at[k0 + 1]], rows1, sem1)
                pltpu.make_async_copy(
                    y_hbm.at[sidx.at[k0]], rows0, sem0).wait()
                pltpu.sync_copy(rows0, acc.at[didx.at[k0]], add=True)

                @pl.when(j < (BB // 2) - 1)
                def _next_even():
                    pltpu.async_copy(y_hbm.at[sidx.at[k0 + 2]], rows0, sem0)

                pltpu.make_async_copy(
                    y_hbm.at[sidx.at[k0 + 1]], rows1, sem1).wait()
                pltpu.sync_copy(rows1, acc.at[didx.at[k0 + 1]], add=True)
                return carry2

            lax.fori_loop(0, BB // 2, pair, 0)
            return carry

        lax.fori_loop(0, NCHT // BB, batch, 0)
        plsc.subcore_barrier()

        pltpu.sync_copy(acc.at[pl.ds(r0, RPT)], part_hbm.at[c, pl.ds(r0, RPT)])

    return agg


_AGG = None


def _get_agg():
    global _AGG
    if _AGG is None:
        _AGG = _make_agg()
    return _AGG


BM = 1000


def _matmul(x, w):
    m, k = x.shape
    n = w.shape[1]

    def body(x_ref, w_ref, o_ref):
        o_ref[...] = jnp.dot(x_ref[...], w_ref[...],
                             preferred_element_type=jnp.float32)

    return pl.pallas_call(
        body,
        grid=(m // BM,),
        in_specs=[pl.BlockSpec((BM, k), lambda i: (i, 0)),
                  pl.BlockSpec((k, n), lambda i: (0, 0))],
        out_specs=pl.BlockSpec((BM, n), lambda i: (i, 0)),
        out_shape=jax.ShapeDtypeStruct((m, n), jnp.float32),
    )(x, w)


def _mid(y, parts, b, w):
    m, n = y.shape

    def body(y_ref, p_ref, b_ref, w_ref, o_ref):
        h = y_ref[...] + p_ref[0] + p_ref[1] + b_ref[...]
        h = jnp.maximum(h, 0.0)
        o_ref[...] = jnp.dot(h, w_ref[...], preferred_element_type=jnp.float32)

    return pl.pallas_call(
        body,
        grid=(m // BM,),
        in_specs=[pl.BlockSpec((BM, n), lambda i: (i, 0)),
                  pl.BlockSpec((NC, BM, n), lambda i: (0, i, 0)),
                  pl.BlockSpec((1, n), lambda i: (0, 0)),
                  pl.BlockSpec((n, n), lambda i: (0, 0))],
        out_specs=pl.BlockSpec((BM, n), lambda i: (i, 0)),
        out_shape=jax.ShapeDtypeStruct((m, n), jnp.float32),
    )(y, parts, b.reshape(1, n), w)


def _final(y, parts, b):
    m, n = y.shape

    def body(y_ref, p_ref, b_ref, o_ref):
        o_ref[...] = jnp.maximum(
            y_ref[...] + p_ref[0] + p_ref[1] + b_ref[...], 0.0)

    return pl.pallas_call(
        body,
        grid=(m // BM,),
        in_specs=[pl.BlockSpec((BM, n), lambda i: (i, 0)),
                  pl.BlockSpec((NC, BM, n), lambda i: (0, i, 0)),
                  pl.BlockSpec((1, n), lambda i: (0, 0))],
        out_specs=pl.BlockSpec((BM, n), lambda i: (i, 0)),
        out_shape=jax.ShapeDtypeStruct((m, n), jnp.float32),
    )(y, parts, b.reshape(1, n))


def kernel(x, edge_index, W1, b1, W2, b2):
    npad = E_PAD - E
    src = jnp.concatenate(
        [edge_index[0],
         (jnp.arange(npad, dtype=jnp.int32) * 13) % N]).reshape(NCHUNK, CH)
    dst = jnp.concatenate(
        [edge_index[1],
         N + (jnp.arange(npad, dtype=jnp.int32) % DUMMY)]).reshape(NCHUNK, CH)
    agg = _get_agg()
    y1 = _matmul(x, W1)
    p1 = agg(y1, src, dst)
    y2 = _mid(y1, p1, b1, W2)
    p2 = agg(y2, src, dst)
    return _final(y2, p2, b2)


# speedup vs baseline: 3.0377x; 1.0023x over previous
import functools

import jax
import jax.numpy as jnp
from jax import lax
from jax.experimental import pallas as pl
from jax.experimental.pallas import tpu as pltpu
from jax.experimental.pallas import tpu_sc as plsc

N = 10000
E = 320000
F = 128
CH = 128
NC = 2
NS = 16
NW = NC * NS
NCHT = 80
NCHUNK = NW * NCHT
E_PAD = NCHUNK * CH
DUMMY = 240
ACC_N = N + DUMMY
RPT = ACC_N // NS
BB = 16
NBUF = 2
ZR = 32


def _make_agg():
    mesh = plsc.VectorSubcoreMesh(core_axis_name="c", subcore_axis_name="s")

    @functools.partial(
        pl.kernel,
        mesh=mesh,
        out_type=jax.ShapeDtypeStruct((NC, ACC_N, F), jnp.float32),
        scratch_types=[
            pltpu.VMEM((BB, CH), jnp.int32),
            pltpu.VMEM((BB, CH), jnp.int32),
            [pltpu.VMEM((CH, F), jnp.float32) for _ in range(NBUF)],
            pltpu.VMEM((ZR, F), jnp.float32),
            pltpu.VMEM_SHARED((ACC_N, F), jnp.float32),
            [pltpu.SemaphoreType.DMA for _ in range(NBUF)],
            [pltpu.SemaphoreType.DMA for _ in range(NBUF)],
        ],
    )
    def agg(y_hbm, src_hbm, dst_hbm, part_hbm,
            sidx, didx, rows, zbuf, acc, gsem, ssem):
        c = lax.axis_index("c")
        s = lax.axis_index("s")
        wid = s * NC + c
        c0 = pl.multiple_of(wid * NCHT, 8)

        def zstore(k, carry):
            i = k // (F // 16)
            j = (k % (F // 16)) * 16
            zbuf[i, pl.ds(j, 16)] = jnp.zeros((16,), jnp.float32)
            return carry

        lax.fori_loop(0, ZR * (F // 16), zstore, 0)
        r0 = pl.multiple_of(s * RPT, 8)

        def zcopy(i, carry):
            pltpu.sync_copy(zbuf, acc.at[pl.ds(r0 + i * ZR, ZR)])
            return carry

        lax.fori_loop(0, RPT // ZR, zcopy, 0)
        plsc.subcore_barrier()

        def g_start(k):
            pltpu.async_copy(y_hbm.at[sidx.at[k]], rows[k % NBUF],
                             gsem[k % NBUF])

        def g_wait(k):
            pltpu.make_async_copy(y_hbm.at[sidx.at[k]], rows[k % NBUF],
                                  gsem[k % NBUF]).wait()

        def s_start(k):
            pltpu.async_copy(rows[k % NBUF], acc.at[didx.at[k]],
                             ssem[k % NBUF], add=True)

        def s_wait(k):
            pltpu.make_async_copy(rows[k % NBUF], acc.at[didx.at[k]],
                                  ssem[k % NBUF]).wait()

        def batch(b, carry):
            cb = pl.multiple_of(c0 + b * BB, 8)
            pltpu.sync_copy(src_hbm.at[pl.ds(cb, BB)], sidx)
            pltpu.sync_copy(dst_hbm.at[pl.ds(cb, BB)], didx)
            for k in range(BB):
                if k >= NBUF:
                    s_wait(k - NBUF)
                g_start(k)
                if k >= 1:
                    g_wait(k - 1)
                    s_start(k - 1)
            g_wait(BB - 1)
            s_start(BB - 1)
            for k in range(BB - NBUF, BB):
                s_wait(k)
            return carry

        lax.fori_loop(0, NCHT // BB, batch, 0)
        plsc.subcore_barrier()

        pltpu.sync_copy(acc.at[pl.ds(r0, RPT)], part_hbm.at[c, pl.ds(r0, RPT)])

    return agg


_AGG = None


def _get_agg():
    global _AGG
    if _AGG is None:
        _AGG = _make_agg()
    return _AGG


BM = 1000


def _matmul(x, w):
    m, k = x.shape
    n = w.shape[1]

    def body(x_ref, w_ref, o_ref):
        o_ref[...] = jnp.dot(x_ref[...], w_ref[...],
                             preferred_element_type=jnp.float32)

    return pl.pallas_call(
        body,
        grid=(m // BM,),
        in_specs=[pl.BlockSpec((BM, k), lambda i: (i, 0)),
                  pl.BlockSpec((k, n), lambda i: (0, 0))],
        out_specs=pl.BlockSpec((BM, n), lambda i: (i, 0)),
        out_shape=jax.ShapeDtypeStruct((m, n), jnp.float32),
    )(x, w)


def _mid(y, parts, b, w):
    m, n = y.shape

    def body(y_ref, p_ref, b_ref, w_ref, o_ref):
        h = y_ref[...] + p_ref[0] + p_ref[1] + b_ref[...]
        h = jnp.maximum(h, 0.0)
        o_ref[...] = jnp.dot(h, w_ref[...], preferred_element_type=jnp.float32)

    return pl.pallas_call(
        body,
        grid=(m // BM,),
        in_specs=[pl.BlockSpec((BM, n), lambda i: (i, 0)),
                  pl.BlockSpec((NC, BM, n), lambda i: (0, i, 0)),
                  pl.BlockSpec((1, n), lambda i: (0, 0)),
                  pl.BlockSpec((n, n), lambda i: (0, 0))],
        out_specs=pl.BlockSpec((BM, n), lambda i: (i, 0)),
        out_shape=jax.ShapeDtypeStruct((m, n), jnp.float32),
    )(y, parts, b.reshape(1, n), w)


def _final(y, parts, b):
    m, n = y.shape

    def body(y_ref, p_ref, b_ref, o_ref):
        o_ref[...] = jnp.maximum(
            y_ref[...] + p_ref[0] + p_ref[1] + b_ref[...], 0.0)

    return pl.pallas_call(
        body,
        grid=(m // BM,),
        in_specs=[pl.BlockSpec((BM, n), lambda i: (i, 0)),
                  pl.BlockSpec((NC, BM, n), lambda i: (0, i, 0)),
                  pl.BlockSpec((1, n), lambda i: (0, 0))],
        out_specs=pl.BlockSpec((BM, n), lambda i: (i, 0)),
        out_shape=jax.ShapeDtypeStruct((m, n), jnp.float32),
    )(y, parts, b.reshape(1, n))


def kernel(x, edge_index, W1, b1, W2, b2):
    npad = E_PAD - E
    src = jnp.concatenate(
        [edge_index[0],
         (jnp.arange(npad, dtype=jnp.int32) * 13) % N]).reshape(NCHUNK, CH)
    dst = jnp.concatenate(
        [edge_index[1],
         N + (jnp.arange(npad, dtype=jnp.int32) % DUMMY)]).reshape(NCHUNK, CH)
    agg = _get_agg()
    y1 = _matmul(x, W1)
    p1 = agg(y1, src, dst)
    y2 = _mid(y1, p1, b1, W2)
    p2 = agg(y2, src, dst)
    return _final(y2, p2, b2)


# speedup vs baseline: 3.0905x; 1.0174x over previous
import functools

import jax
import jax.numpy as jnp
from jax import lax
from jax.experimental import pallas as pl
from jax.experimental.pallas import tpu as pltpu
from jax.experimental.pallas import tpu_sc as plsc

N = 10000
E = 320000
F = 128
CH = 128
NC = 2
NS = 16
NW = NC * NS
NCHT = 80
NCHUNK = NW * NCHT
E_PAD = NCHUNK * CH
DUMMY = 240
ACC_N = N + DUMMY
RPT = ACC_N // NS
BB = 16
NBUF = 2
ZR = 32


def _make_agg():
    mesh = plsc.VectorSubcoreMesh(core_axis_name="c", subcore_axis_name="s")

    @functools.partial(
        pl.kernel,
        mesh=mesh,
        out_type=jax.ShapeDtypeStruct((NC, ACC_N, F), jnp.float32),
        scratch_types=[
            pltpu.VMEM((BB, CH), jnp.int32),
            pltpu.VMEM((BB, CH), jnp.int32),
            [pltpu.VMEM((CH, F), jnp.float32) for _ in range(NBUF)],
            pltpu.VMEM((ZR, F), jnp.float32),
            pltpu.VMEM_SHARED((ACC_N, F), jnp.float32),
            [pltpu.SemaphoreType.DMA for _ in range(NBUF)],
            [pltpu.SemaphoreType.DMA for _ in range(NBUF)],
        ],
    )
    def agg(y_hbm, src_hbm, dst_hbm, part_hbm,
            sidx, didx, rows, zbuf, acc, gsem, ssem):
        c = lax.axis_index("c")
        s = lax.axis_index("s")
        wid = s * NC + c
        c0 = pl.multiple_of(wid * NCHT, 8)

        def zstore(k, carry):
            i = k // (F // 16)
            j = (k % (F // 16)) * 16
            zbuf[i, pl.ds(j, 16)] = jnp.zeros((16,), jnp.float32)
            return carry

        lax.fori_loop(0, ZR * (F // 16), zstore, 0)
        r0 = pl.multiple_of(s * RPT, 8)

        def zcopy(i, carry):
            pltpu.async_copy(zbuf, acc.at[pl.ds(r0 + i * ZR, ZR)], ssem[0])
            return carry

        lax.fori_loop(0, RPT // ZR, zcopy, 0)
        pltpu.sync_copy(src_hbm.at[pl.ds(c0, BB)], sidx)
        pltpu.sync_copy(dst_hbm.at[pl.ds(c0, BB)], didx)

        def zdrain(i, carry):
            pltpu.make_async_copy(zbuf, acc.at[pl.ds(r0 + i * ZR, ZR)],
                                  ssem[0]).wait()
            return carry

        lax.fori_loop(0, RPT // ZR, zdrain, 0)
        plsc.subcore_barrier()

        def g_start(k):
            pltpu.async_copy(y_hbm.at[sidx.at[k]], rows[k % NBUF],
                             gsem[k % NBUF])

        def g_wait(k):
            pltpu.make_async_copy(y_hbm.at[sidx.at[k]], rows[k % NBUF],
                                  gsem[k % NBUF]).wait()

        def s_start(k):
            pltpu.async_copy(rows[k % NBUF], acc.at[didx.at[k]],
                             ssem[k % NBUF], add=True)

        def s_wait(k):
            pltpu.make_async_copy(rows[k % NBUF], acc.at[didx.at[k]],
                                  ssem[k % NBUF]).wait()

        def batch(b, carry):
            cb = pl.multiple_of(c0 + b * BB, 8)

            @pl.when(b > 0)
            def _load_idx():
                pltpu.sync_copy(src_hbm.at[pl.ds(cb, BB)], sidx)
                pltpu.sync_copy(dst_hbm.at[pl.ds(cb, BB)], didx)
            for k in range(BB):
                if k >= NBUF:
                    s_wait(k - NBUF)
                g_start(k)
                if k >= 1:
                    g_wait(k - 1)
                    s_start(k - 1)
            g_wait(BB - 1)
            s_start(BB - 1)
            for k in range(BB - NBUF, BB):
                s_wait(k)
            return carry

        lax.fori_loop(0, NCHT // BB, batch, 0)
        plsc.subcore_barrier()

        pltpu.sync_copy(acc.at[pl.ds(r0, RPT)], part_hbm.at[c, pl.ds(r0, RPT)])

    return agg


_AGG = None


def _get_agg():
    global _AGG
    if _AGG is None:
        _AGG = _make_agg()
    return _AGG


BM = 1000


def _matmul(x, w):
    m, k = x.shape
    n = w.shape[1]

    def body(x_ref, w_ref, o_ref):
        o_ref[...] = jnp.dot(x_ref[...], w_ref[...],
                             preferred_element_type=jnp.float32)

    return pl.pallas_call(
        body,
        grid=(m // BM,),
        in_specs=[pl.BlockSpec((BM, k), lambda i: (i, 0)),
                  pl.BlockSpec((k, n), lambda i: (0, 0))],
        out_specs=pl.BlockSpec((BM, n), lambda i: (i, 0)),
        out_shape=jax.ShapeDtypeStruct((m, n), jnp.float32),
    )(x, w)


def _mid(y, parts, b, w):
    m, n = y.shape

    def body(y_ref, p_ref, b_ref, w_ref, o_ref):
        h = y_ref[...] + p_ref[0] + p_ref[1] + b_ref[...]
        h = jnp.maximum(h, 0.0)
        o_ref[...] = jnp.dot(h, w_ref[...], preferred_element_type=jnp.float32)

    return pl.pallas_call(
        body,
        grid=(m // BM,),
        in_specs=[pl.BlockSpec((BM, n), lambda i: (i, 0)),
                  pl.BlockSpec((NC, BM, n), lambda i: (0, i, 0)),
                  pl.BlockSpec((1, n), lambda i: (0, 0)),
                  pl.BlockSpec((n, n), lambda i: (0, 0))],
        out_specs=pl.BlockSpec((BM, n), lambda i: (i, 0)),
        out_shape=jax.ShapeDtypeStruct((m, n), jnp.float32),
    )(y, parts, b.reshape(1, n), w)


def _final(y, parts, b):
    m, n = y.shape

    def body(y_ref, p_ref, b_ref, o_ref):
        o_ref[...] = jnp.maximum(
            y_ref[...] + p_ref[0] + p_ref[1] + b_ref[...], 0.0)

    return pl.pallas_call(
        body,
        grid=(m // BM,),
        in_specs=[pl.BlockSpec((BM, n), lambda i: (i, 0)),
                  pl.BlockSpec((NC, BM, n), lambda i: (0, i, 0)),
                  pl.BlockSpec((1, n), lambda i: (0, 0))],
        out_specs=pl.BlockSpec((BM, n), lambda i: (i, 0)),
        out_shape=jax.ShapeDtypeStruct((m, n), jnp.float32),
    )(y, parts, b.reshape(1, n))


def kernel(x, edge_index, W1, b1, W2, b2):
    npad = E_PAD - E
    src = jnp.concatenate(
        [edge_index[0],
         (jnp.arange(npad, dtype=jnp.int32) * 13) % N]).reshape(NCHUNK, CH)
    dst = jnp.concatenate(
        [edge_index[1],
         N + (jnp.arange(npad, dtype=jnp.int32) % DUMMY)]).reshape(NCHUNK, CH)
    agg = _get_agg()
    y1 = _matmul(x, W1)
    p1 = agg(y1, src, dst)
    y2 = _mid(y1, p1, b1, W2)
    p2 = agg(y2, src, dst)
    return _final(y2, p2, b2)


# speedup vs baseline: 3.2675x; 1.0573x over previous
import functools

import jax
import jax.numpy as jnp
from jax import lax
from jax.experimental import pallas as pl
from jax.experimental.pallas import tpu as pltpu
from jax.experimental.pallas import tpu_sc as plsc

N = 10000
E = 320000
F = 128
CH = 128
NC = 2
NS = 16
NW = NC * NS
NCHT = 80
NCHUNK = NW * NCHT
E_PAD = NCHUNK * CH
DUMMY = 240
ACC_N = N + DUMMY
RPT = ACC_N // NS
BB = 40
NBUF = 2
ZR = 32


def _make_agg():
    mesh = plsc.VectorSubcoreMesh(core_axis_name="c", subcore_axis_name="s")

    @functools.partial(
        pl.kernel,
        mesh=mesh,
        out_type=jax.ShapeDtypeStruct((NC, ACC_N, F), jnp.float32),
        scratch_types=[
            pltpu.VMEM((BB, CH), jnp.int32),
            pltpu.VMEM((BB, CH), jnp.int32),
            [pltpu.VMEM((CH, F), jnp.float32) for _ in range(NBUF)],
            pltpu.VMEM((ZR, F), jnp.float32),
            pltpu.VMEM_SHARED((ACC_N, F), jnp.float32),
            [pltpu.SemaphoreType.DMA for _ in range(NBUF)],
            [pltpu.SemaphoreType.DMA for _ in range(NBUF)],
        ],
    )
    def agg(y_hbm, src_hbm, dst_hbm, part_hbm,
            sidx, didx, rows, zbuf, acc, gsem, ssem):
        c = lax.axis_index("c")
        s = lax.axis_index("s")
        wid = s * NC + c
        c0 = pl.multiple_of(wid * NCHT, 8)

        def zstore(k, carry):
            i = k // (F // 16)
            j = (k % (F // 16)) * 16
            zbuf[i, pl.ds(j, 16)] = jnp.zeros((16,), jnp.float32)
            return carry

        lax.fori_loop(0, ZR * (F // 16), zstore, 0)
        r0 = pl.multiple_of(s * RPT, 8)

        def zcopy(i, carry):
            pltpu.async_copy(zbuf, acc.at[pl.ds(r0 + i * ZR, ZR)], ssem[0])
            return carry

        lax.fori_loop(0, RPT // ZR, zcopy, 0)
        pltpu.sync_copy(src_hbm.at[pl.ds(c0, BB)], sidx)
        pltpu.sync_copy(dst_hbm.at[pl.ds(c0, BB)], didx)

        def zdrain(i, carry):
            pltpu.make_async_copy(zbuf, acc.at[pl.ds(r0 + i * ZR, ZR)],
                                  ssem[0]).wait()
            return carry

        lax.fori_loop(0, RPT // ZR, zdrain, 0)
        plsc.subcore_barrier()

        def g_start(k):
            pltpu.async_copy(y_hbm.at[sidx.at[k]], rows[k % NBUF],
                             gsem[k % NBUF])

        def g_wait(k):
            pltpu.make_async_copy(y_hbm.at[sidx.at[k]], rows[k % NBUF],
                                  gsem[k % NBUF]).wait()

        def s_start(k):
            pltpu.async_copy(rows[k % NBUF], acc.at[didx.at[k]],
                             ssem[k % NBUF], add=True)

        def s_wait(k):
            pltpu.make_async_copy(rows[k % NBUF], acc.at[didx.at[k]],
                                  ssem[k % NBUF]).wait()

        def batch(b, carry):
            cb = pl.multiple_of(c0 + b * BB, 8)

            @pl.when(b > 0)
            def _load_idx():
                pltpu.sync_copy(src_hbm.at[pl.ds(cb, BB)], sidx)
                pltpu.sync_copy(dst_hbm.at[pl.ds(cb, BB)], didx)
            for k in range(BB):
                if k >= NBUF:
                    s_wait(k - NBUF)
                g_start(k)
                if k >= 1:
                    g_wait(k - 1)
                    s_start(k - 1)
            g_wait(BB - 1)
            s_start(BB - 1)
            for k in range(BB - NBUF, BB):
                s_wait(k)
            return carry

        lax.fori_loop(0, NCHT // BB, batch, 0)
        plsc.subcore_barrier()

        pltpu.sync_copy(acc.at[pl.ds(r0, RPT)], part_hbm.at[c, pl.ds(r0, RPT)])

    return agg


_AGG = None


def _get_agg():
    global _AGG
    if _AGG is None:
        _AGG = _make_agg()
    return _AGG


BM = 1000


def _matmul(x, w):
    m, k = x.shape
    n = w.shape[1]

    def body(x_ref, w_ref, o_ref):
        o_ref[...] = jnp.dot(x_ref[...], w_ref[...],
                             preferred_element_type=jnp.float32)

    return pl.pallas_call(
        body,
        grid=(m // BM,),
        in_specs=[pl.BlockSpec((BM, k), lambda i: (i, 0)),
                  pl.BlockSpec((k, n), lambda i: (0, 0))],
        out_specs=pl.BlockSpec((BM, n), lambda i: (i, 0)),
        out_shape=jax.ShapeDtypeStruct((m, n), jnp.float32),
    )(x, w)


def _mid(y, parts, b, w):
    m, n = y.shape

    def body(y_ref, p_ref, b_ref, w_ref, o_ref):
        h = y_ref[...] + p_ref[0] + p_ref[1] + b_ref[...]
        h = jnp.maximum(h, 0.0)
        o_ref[...] = jnp.dot(h, w_ref[...], preferred_element_type=jnp.float32)

    return pl.pallas_call(
        body,
        grid=(m // BM,),
        in_specs=[pl.BlockSpec((BM, n), lambda i: (i, 0)),
                  pl.BlockSpec((NC, BM, n), lambda i: (0, i, 0)),
                  pl.BlockSpec((1, n), lambda i: (0, 0)),
                  pl.BlockSpec((n, n), lambda i: (0, 0))],
        out_specs=pl.BlockSpec((BM, n), lambda i: (i, 0)),
        out_shape=jax.ShapeDtypeStruct((m, n), jnp.float32),
    )(y, parts, b.reshape(1, n), w)


def _final(y, parts, b):
    m, n = y.shape

    def body(y_ref, p_ref, b_ref, o_ref):
        o_ref[...] = jnp.maximum(
            y_ref[...] + p_ref[0] + p_ref[1] + b_ref[...], 0.0)

    return pl.pallas_call(
        body,
        grid=(m // BM,),
        in_specs=[pl.BlockSpec((BM, n), lambda i: (i, 0)),
                  pl.BlockSpec((NC, BM, n), lambda i: (0, i, 0)),
                  pl.BlockSpec((1, n), lambda i: (0, 0))],
        out_specs=pl.BlockSpec((BM, n), lambda i: (i, 0)),
        out_shape=jax.ShapeDtypeStruct((m, n), jnp.float32),
    )(y, parts, b.reshape(1, n))


def kernel(x, edge_index, W1, b1, W2, b2):
    npad = E_PAD - E
    src = jnp.concatenate(
        [edge_index[0],
         (jnp.arange(npad, dtype=jnp.int32) * 13) % N]).reshape(NCHUNK, CH)
    dst = jnp.concatenate(
        [edge_index[1],
         N + (jnp.arange(npad, dtype=jnp.int32) % DUMMY)]).reshape(NCHUNK, CH)
    agg = _get_agg()
    y1 = _matmul(x, W1)
    p1 = agg(y1, src, dst)
    y2 = _mid(y1, p1, b1, W2)
    p2 = agg(y2, src, dst)
    return _final(y2, p2, b2)


# speedup vs baseline: 3.3501x; 1.0253x over previous
import functools

import jax
import jax.numpy as jnp
from jax import lax
from jax.experimental import pallas as pl
from jax.experimental.pallas import tpu as pltpu
from jax.experimental.pallas import tpu_sc as plsc

N = 10000
E = 320000
F = 128
CH = 128
NC = 2
NS = 16
NW = NC * NS
NCHT = 80
NCHUNK = NW * NCHT
E_PAD = NCHUNK * CH
DUMMY = 240
ACC_N = N + DUMMY
RPT = ACC_N // NS
BB = 40
NBUF = 2
ZR = 32


def _make_agg():
    mesh = plsc.VectorSubcoreMesh(core_axis_name="c", subcore_axis_name="s")

    @functools.partial(
        pl.kernel,
        mesh=mesh,
        out_type=jax.ShapeDtypeStruct((NC, ACC_N, F), jnp.float32),
        scratch_types=[
            pltpu.VMEM((BB, CH), jnp.int32),
            pltpu.VMEM((BB, CH), jnp.int32),
            [pltpu.VMEM((CH, F), jnp.float32) for _ in range(NBUF)],
            pltpu.VMEM((ZR, F), jnp.float32),
            pltpu.VMEM_SHARED((ACC_N, F), jnp.float32),
            [pltpu.SemaphoreType.DMA for _ in range(NBUF)],
            [pltpu.SemaphoreType.DMA for _ in range(NBUF)],
        ],
    )
    def agg(y_hbm, src_hbm, dst_hbm, part_hbm,
            sidx, didx, rows, zbuf, acc, gsem, ssem):
        c = lax.axis_index("c")
        s = lax.axis_index("s")
        wid = s * NC + c
        c0 = pl.multiple_of(wid * NCHT, 8)

        def zstore(k, carry):
            i = k // (F // 16)
            j = (k % (F // 16)) * 16
            zbuf[i, pl.ds(j, 16)] = jnp.zeros((16,), jnp.float32)
            return carry

        lax.fori_loop(0, ZR * (F // 16), zstore, 0)
        r0 = pl.multiple_of(s * RPT, 8)

        def zcopy(i, carry):
            pltpu.async_copy(zbuf, acc.at[pl.ds(r0 + i * ZR, ZR)], ssem[0])
            return carry

        lax.fori_loop(0, RPT // ZR, zcopy, 0)
        pltpu.sync_copy(src_hbm.at[pl.ds(c0, BB)], sidx)
        pltpu.sync_copy(dst_hbm.at[pl.ds(c0, BB)], didx)

        def zdrain(i, carry):
            pltpu.make_async_copy(zbuf, acc.at[pl.ds(r0 + i * ZR, ZR)],
                                  ssem[0]).wait()
            return carry

        lax.fori_loop(0, RPT // ZR, zdrain, 0)
        plsc.subcore_barrier()

        def g_start(k):
            pltpu.async_copy(y_hbm.at[sidx.at[k]], rows[k % NBUF],
                             gsem[k % NBUF])

        def g_wait(k):
            pltpu.make_async_copy(y_hbm.at[sidx.at[k]], rows[k % NBUF],
                                  gsem[k % NBUF]).wait()

        def s_start(k):
            pltpu.async_copy(rows[k % NBUF], acc.at[didx.at[k]],
                             ssem[k % NBUF], add=True)

        def s_wait(k):
            pltpu.make_async_copy(rows[k % NBUF], acc.at[didx.at[k]],
                                  ssem[k % NBUF]).wait()

        def batch(b, carry):
            cb = pl.multiple_of(c0 + b * BB, 8)

            @pl.when(b > 0)
            def _load_idx():
                pltpu.sync_copy(src_hbm.at[pl.ds(cb, BB)], sidx)
                pltpu.sync_copy(dst_hbm.at[pl.ds(cb, BB)], didx)
            for k in range(BB):
                if k >= NBUF:
                    s_wait(k - NBUF)
                g_start(k)
                if k >= 1:
                    g_wait(k - 1)
                    s_start(k - 1)
            g_wait(BB - 1)
            s_start(BB - 1)
            for k in range(BB - NBUF, BB):
                s_wait(k)
            return carry

        lax.fori_loop(0, NCHT // BB, batch, 0)
        plsc.subcore_barrier()

        pltpu.sync_copy(acc.at[pl.ds(r0, RPT)], part_hbm.at[c, pl.ds(r0, RPT)])

    return agg


_AGG = None


def _get_agg():
    global _AGG
    if _AGG is None:
        _AGG = _make_agg()
    return _AGG


BM = 2000


def _matmul(x, w):
    m, k = x.shape
    n = w.shape[1]

    def body(x_ref, w_ref, o_ref):
        o_ref[...] = jnp.dot(x_ref[...], w_ref[...],
                             preferred_element_type=jnp.float32)

    return pl.pallas_call(
        body,
        grid=(m // BM,),
        in_specs=[pl.BlockSpec((BM, k), lambda i: (i, 0)),
                  pl.BlockSpec((k, n), lambda i: (0, 0))],
        out_specs=pl.BlockSpec((BM, n), lambda i: (i, 0)),
        out_shape=jax.ShapeDtypeStruct((m, n), jnp.float32),
    )(x, w)


def _mid(y, parts, b, w):
    m, n = y.shape

    def body(y_ref, p_ref, b_ref, w_ref, o_ref):
        h = y_ref[...] + p_ref[0] + p_ref[1] + b_ref[...]
        h = jnp.maximum(h, 0.0)
        o_ref[...] = jnp.dot(h, w_ref[...], preferred_element_type=jnp.float32)

    return pl.pallas_call(
        body,
        grid=(m // BM,),
        in_specs=[pl.BlockSpec((BM, n), lambda i: (i, 0)),
                  pl.BlockSpec((NC, BM, n), lambda i: (0, i, 0)),
                  pl.BlockSpec((1, n), lambda i: (0, 0)),
                  pl.BlockSpec((n, n), lambda i: (0, 0))],
        out_specs=pl.BlockSpec((BM, n), lambda i: (i, 0)),
        out_shape=jax.ShapeDtypeStruct((m, n), jnp.float32),
    )(y, parts, b.reshape(1, n), w)


def _final(y, parts, b):
    m, n = y.shape

    def body(y_ref, p_ref, b_ref, o_ref):
        o_ref[...] = jnp.maximum(
            y_ref[...] + p_ref[0] + p_ref[1] + b_ref[...], 0.0)

    return pl.pallas_call(
        body,
        grid=(m // BM,),
        in_specs=[pl.BlockSpec((BM, n), lambda i: (i, 0)),
                  pl.BlockSpec((NC, BM, n), lambda i: (0, i, 0)),
                  pl.BlockSpec((1, n), lambda i: (0, 0))],
        out_specs=pl.BlockSpec((BM, n), lambda i: (i, 0)),
        out_shape=jax.ShapeDtypeStruct((m, n), jnp.float32),
    )(y, parts, b.reshape(1, n))


def kernel(x, edge_index, W1, b1, W2, b2):
    npad = E_PAD - E
    src = jnp.concatenate(
        [edge_index[0],
         (jnp.arange(npad, dtype=jnp.int32) * 13) % N]).reshape(NCHUNK, CH)
    dst = jnp.concatenate(
        [edge_index[1],
         N + (jnp.arange(npad, dtype=jnp.int32) % DUMMY)]).reshape(NCHUNK, CH)
    agg = _get_agg()
    y1 = _matmul(x, W1)
    p1 = agg(y1, src, dst)
    y2 = _mid(y1, p1, b1, W2)
    p2 = agg(y2, src, dst)
    return _final(y2, p2, b2)
